# Initial kernel scaffold; baseline (speedup 1.0000x reference)
import jax
import jax.numpy as jnp
from jax.experimental import pallas as pl


def kernel(x, edge_index, W1, b1, W2, b2):
    raise NotImplementedError("write your pallas kernel here")


# speedup vs baseline: 55.2231x; 55.2231x over previous
import functools

import jax
import jax.numpy as jnp
from jax import lax
from jax.experimental import pallas as pl
from jax.experimental.pallas import tpu as pltpu
from jax.experimental.pallas import tpu_sc as plsc

NC = 2
NS = 16
NW = NC * NS
LANES = 128
KROWS = 8
CHUNK = KROWS * LANES
DEPTH = 8
TB = 1024


def _geom(n_nodes, n_edges):
    np_pad = ((n_nodes + 1 + TB - 1) // TB) * TB
    nblk = ((n_edges + CHUNK - 1) // CHUNK + NW - 1) // NW * NW
    return np_pad, nblk


@functools.lru_cache(maxsize=None)
def _sc_scatter_kernel(np_pad, nblk, gather):
    bpw = nblk // NW
    rpt = np_pad // NS
    mesh = plsc.VectorSubcoreMesh(
        core_axis_name="c", subcore_axis_name="s",
        num_cores=NC, num_subcores=NS)
    out_type = (jax.ShapeDtypeStruct((np_pad, DEPTH), jnp.float32),
                jax.ShapeDtypeStruct((np_pad, DEPTH), jnp.float32))
    scratch = [
        [pltpu.VMEM((LANES,), jnp.int32) for _ in range(KROWS)],
        [pltpu.VMEM((LANES,), jnp.int32) for _ in range(KROWS)],
        [pltpu.VMEM((LANES, DEPTH), jnp.float32) for _ in range(KROWS)],
        pltpu.VMEM_SHARED((np_pad, DEPTH), jnp.float32),
        pltpu.SemaphoreType.DMA,
        pltpu.SemaphoreType.DMA,
        pltpu.SemaphoreType.DMA,
    ]

    if gather:
        def body(src2, dst2, table, zeros, out0, out1,
                 src_v, dst_v, rows_v, agg, isem, gsem, ssem):
            cid = lax.axis_index("c")
            sid = lax.axis_index("s")
            wid = sid * NC + cid
            stripe = pl.ds(sid * rpt, rpt)
            pltpu.sync_copy(zeros.at[stripe], agg.at[stripe])
            plsc.subcore_barrier()
            base = wid * bpw

            def step(i, carry):
                row0 = (base + i) * KROWS
                loads = [pltpu.async_copy(src2.at[row0 + j], src_v[j], isem)
                         for j in range(KROWS)]
                loads += [pltpu.async_copy(dst2.at[row0 + j], dst_v[j], isem)
                          for j in range(KROWS)]
                for ld in loads:
                    ld.wait()
                gathers = [
                    pltpu.async_copy(table.at[src_v[j]], rows_v[j], gsem)
                    for j in range(KROWS)]
                for g in gathers:
                    g.wait()
                scatters = [
                    pltpu.async_copy(rows_v[j], agg.at[dst_v[j]], ssem,
                                     add=True)
                    for j in range(KROWS)]
                for s in scatters:
                    s.wait()
                return carry

            lax.fori_loop(0, bpw, step, 0)
            plsc.subcore_barrier()

            @pl.when(cid == 0)
            def _():
                pltpu.sync_copy(agg.at[stripe], out0.at[stripe])

            @pl.when(cid == 1)
            def _():
                pltpu.sync_copy(agg.at[stripe], out1.at[stripe])
    else:
        def body(dst2, ones, zeros, out0, out1,
                 src_v, dst_v, rows_v, agg, isem, gsem, ssem):
            cid = lax.axis_index("c")
            sid = lax.axis_index("s")
            wid = sid * NC + cid
            stripe = pl.ds(sid * rpt, rpt)
            pltpu.sync_copy(zeros.at[stripe], agg.at[stripe])
            for j in range(KROWS):
                pltpu.sync_copy(ones, rows_v[j])
            plsc.subcore_barrier()
            base = wid * bpw

            def step(i, carry):
                row0 = (base + i) * KROWS
                loads = [pltpu.async_copy(dst2.at[row0 + j], dst_v[j], isem)
                         for j in range(KROWS)]
                for ld in loads:
                    ld.wait()
                scatters = [
                    pltpu.async_copy(rows_v[j], agg.at[dst_v[j]], ssem,
                                     add=True)
                    for j in range(KROWS)]
                for s in scatters:
                    s.wait()
                return carry

            lax.fori_loop(0, bpw, step, 0)
            plsc.subcore_barrier()

            @pl.when(cid == 0)
            def _():
                pltpu.sync_copy(agg.at[stripe], out0.at[stripe])

            @pl.when(cid == 1)
            def _():
                pltpu.sync_copy(agg.at[stripe], out1.at[stripe])

    return pl.kernel(
        body, out_type=out_type, mesh=mesh, scratch_types=scratch,
        compiler_params=pltpu.CompilerParams(use_tc_tiling_on_sc=False))


def _row_spec(d):
    return pl.BlockSpec((TB, d), lambda i: (i, 0))


def _full_spec(shape):
    return pl.BlockSpec(shape, lambda i: (0, 0))


def _tc_a(np_pad, deg0, deg1, xp):
    def body(d0, d1, x, dis_o, y1_o):
        dis = lax.rsqrt(d0[...] + d1[...] + 1.0)
        dis_o[...] = dis
        y1_o[...] = x[...] * dis

    return pl.pallas_call(
        body,
        grid=(np_pad // TB,),
        in_specs=[_row_spec(1), _row_spec(1), _row_spec(DEPTH)],
        out_specs=[_row_spec(1), _row_spec(DEPTH)],
        out_shape=[jax.ShapeDtypeStruct((np_pad, 1), jnp.float32),
                   jax.ShapeDtypeStruct((np_pad, DEPTH), jnp.float32)],
    )(deg0, deg1, xp)


def _tc_b(np_pad, a0, a1, y1, dis, W1p, b1, W2p):
    def body(a0_r, a1_r, y1_r, dis_r, w1_r, b1_r, w2_r, y2_o):
        t = (a0_r[...] + a1_r[...] + y1_r[...]) * dis_r[...]
        h1 = jnp.dot(t, w1_r[...], preferred_element_type=jnp.float32)
        h1 = jnp.maximum(h1 + b1_r[...], 0.0)
        y2_o[...] = jnp.dot(
            h1, w2_r[...], preferred_element_type=jnp.float32) * dis_r[...]

    return pl.pallas_call(
        body,
        grid=(np_pad // TB,),
        in_specs=[_row_spec(DEPTH), _row_spec(DEPTH), _row_spec(DEPTH),
                  _row_spec(1),
                  _full_spec((DEPTH, 16)), _full_spec((1, 16)),
                  _full_spec((16, DEPTH))],
        out_specs=_row_spec(DEPTH),
        out_shape=jax.ShapeDtypeStruct((np_pad, DEPTH), jnp.float32),
    )(a0, a1, y1, dis, W1p, b1, W2p)


def _tc_c(np_pad, g0, g1, y2, dis, b2):
    def body(g0_r, g1_r, y2_r, dis_r, b2_r, out_o):
        t = (g0_r[...] + g1_r[...] + y2_r[...]) * dis_r[...]
        out_o[...] = t[:, :2] + b2_r[...]

    return pl.pallas_call(
        body,
        grid=(np_pad // TB,),
        in_specs=[_row_spec(DEPTH), _row_spec(DEPTH), _row_spec(DEPTH),
                  _row_spec(1), _full_spec((1, 2))],
        out_specs=_row_spec(2),
        out_shape=jax.ShapeDtypeStruct((np_pad, 2), jnp.float32),
    )(g0, g1, y2, dis, b2)


def kernel(x, edge_index, W1, b1, W2, b2):
    n_nodes = x.shape[0]
    n_edges = edge_index.shape[1]
    np_pad, nblk = _geom(n_nodes, n_edges)
    e_pad = nblk * CHUNK

    fill = jnp.full((e_pad - n_edges,), n_nodes, dtype=jnp.int32)
    src2 = jnp.concatenate([edge_index[0], fill]).reshape(-1, LANES)
    dst2 = jnp.concatenate([edge_index[1], fill]).reshape(-1, LANES)
    xp = jnp.pad(x, ((0, np_pad - n_nodes), (0, DEPTH - x.shape[1])))
    ones_k = jnp.ones((LANES, DEPTH), jnp.float32)
    zeros_t = jnp.zeros((np_pad, DEPTH), jnp.float32)
    W1p = jnp.pad(W1, ((0, DEPTH - W1.shape[0]), (0, 0)))
    W2p = jnp.pad(W2, ((0, 0), (0, DEPTH - W2.shape[1])))

    deg = _sc_scatter_kernel(np_pad, nblk, False)(dst2, ones_k, zeros_t)
    dis, y1 = _tc_a(np_pad, deg[0][:, :1], deg[1][:, :1], xp)
    agg1 = _sc_scatter_kernel(np_pad, nblk, True)(src2, dst2, y1, zeros_t)
    y2 = _tc_b(np_pad, agg1[0], agg1[1], y1, dis,
               W1p, b1.reshape(1, 16), W2p)
    agg2 = _sc_scatter_kernel(np_pad, nblk, True)(src2, dst2, y2, zeros_t)
    out = _tc_c(np_pad, agg2[0], agg2[1], y2, dis, b2.reshape(1, 2))
    return out[:n_nodes]


# speedup vs baseline: 59.3452x; 1.0746x over previous
import functools

import jax
import jax.numpy as jnp
from jax import lax
from jax.experimental import pallas as pl
from jax.experimental.pallas import tpu as pltpu
from jax.experimental.pallas import tpu_sc as plsc

NC = 2
NS = 16
NW = NC * NS
LANES = 128
KROWS = 16
CHUNK = KROWS * LANES
DEPTH = 8
TB = 1024


def _geom(n_nodes, n_edges):
    np_pad = ((n_nodes + 1 + TB - 1) // TB) * TB
    nblk = ((n_edges + CHUNK - 1) // CHUNK + NW - 1) // NW * NW
    return np_pad, nblk


@functools.lru_cache(maxsize=None)
def _sc_scatter_kernel(np_pad, nblk, gather):
    bpw = nblk // NW
    rpt = np_pad // NS
    mesh = plsc.VectorSubcoreMesh(
        core_axis_name="c", subcore_axis_name="s",
        num_cores=NC, num_subcores=NS)
    out_type = (jax.ShapeDtypeStruct((np_pad, DEPTH), jnp.float32),
                jax.ShapeDtypeStruct((np_pad, DEPTH), jnp.float32))
    scratch = [
        [pltpu.VMEM((LANES,), jnp.int32) for _ in range(KROWS)],
        [pltpu.VMEM((LANES,), jnp.int32) for _ in range(KROWS)],
        [pltpu.VMEM((LANES, DEPTH), jnp.float32) for _ in range(KROWS)],
        pltpu.VMEM_SHARED((np_pad, DEPTH), jnp.float32),
        pltpu.SemaphoreType.DMA,
        pltpu.SemaphoreType.DMA,
        pltpu.SemaphoreType.DMA,
    ]

    if gather:
        def body(src2, dst2, table, zeros, out0, out1,
                 src_v, dst_v, rows_v, agg, isem, gsem, ssem):
            cid = lax.axis_index("c")
            sid = lax.axis_index("s")
            wid = sid * NC + cid
            stripe = pl.ds(sid * rpt, rpt)
            pltpu.sync_copy(zeros.at[stripe], agg.at[stripe])
            plsc.subcore_barrier()
            base = wid * bpw

            def step(i, carry):
                row0 = (base + i) * KROWS
                loads = [pltpu.async_copy(src2.at[row0 + j], src_v[j], isem)
                         for j in range(KROWS)]
                loads += [pltpu.async_copy(dst2.at[row0 + j], dst_v[j], isem)
                          for j in range(KROWS)]
                for ld in loads:
                    ld.wait()
                gathers = [
                    pltpu.async_copy(table.at[src_v[j]], rows_v[j], gsem)
                    for j in range(KROWS)]
                for g in gathers:
                    g.wait()
                scatters = [
                    pltpu.async_copy(rows_v[j], agg.at[dst_v[j]], ssem,
                                     add=True)
                    for j in range(KROWS)]
                for s in scatters:
                    s.wait()
                return carry

            lax.fori_loop(0, bpw, step, 0)
            plsc.subcore_barrier()

            @pl.when(cid == 0)
            def _():
                pltpu.sync_copy(agg.at[stripe], out0.at[stripe])

            @pl.when(cid == 1)
            def _():
                pltpu.sync_copy(agg.at[stripe], out1.at[stripe])
    else:
        def body(dst2, ones, zeros, out0, out1,
                 src_v, dst_v, rows_v, agg, isem, gsem, ssem):
            cid = lax.axis_index("c")
            sid = lax.axis_index("s")
            wid = sid * NC + cid
            stripe = pl.ds(sid * rpt, rpt)
            pltpu.sync_copy(zeros.at[stripe], agg.at[stripe])
            for j in range(KROWS):
                pltpu.sync_copy(ones, rows_v[j])
            plsc.subcore_barrier()
            base = wid * bpw

            def step(i, carry):
                row0 = (base + i) * KROWS
                loads = [pltpu.async_copy(dst2.at[row0 + j], dst_v[j], isem)
                         for j in range(KROWS)]
                for ld in loads:
                    ld.wait()
                scatters = [
                    pltpu.async_copy(rows_v[j], agg.at[dst_v[j]], ssem,
                                     add=True)
                    for j in range(KROWS)]
                for s in scatters:
                    s.wait()
                return carry

            lax.fori_loop(0, bpw, step, 0)
            plsc.subcore_barrier()

            @pl.when(cid == 0)
            def _():
                pltpu.sync_copy(agg.at[stripe], out0.at[stripe])

            @pl.when(cid == 1)
            def _():
                pltpu.sync_copy(agg.at[stripe], out1.at[stripe])

    return pl.kernel(
        body, out_type=out_type, mesh=mesh, scratch_types=scratch,
        compiler_params=pltpu.CompilerParams(use_tc_tiling_on_sc=False))


def _row_spec(d):
    return pl.BlockSpec((TB, d), lambda i: (i, 0))


def _full_spec(shape):
    return pl.BlockSpec(shape, lambda i: (0, 0))


def _tc_a(np_pad, deg0, deg1, xp):
    def body(d0, d1, x, dis_o, y1_o):
        dis = lax.rsqrt(d0[...] + d1[...] + 1.0)
        dis_o[...] = dis
        y1_o[...] = x[...] * dis

    return pl.pallas_call(
        body,
        grid=(np_pad // TB,),
        in_specs=[_row_spec(1), _row_spec(1), _row_spec(DEPTH)],
        out_specs=[_row_spec(1), _row_spec(DEPTH)],
        out_shape=[jax.ShapeDtypeStruct((np_pad, 1), jnp.float32),
                   jax.ShapeDtypeStruct((np_pad, DEPTH), jnp.float32)],
    )(deg0, deg1, xp)


def _tc_b(np_pad, a0, a1, y1, dis, W1p, b1, W2p):
    def body(a0_r, a1_r, y1_r, dis_r, w1_r, b1_r, w2_r, y2_o):
        t = (a0_r[...] + a1_r[...] + y1_r[...]) * dis_r[...]
        h1 = jnp.dot(t, w1_r[...], preferred_element_type=jnp.float32)
        h1 = jnp.maximum(h1 + b1_r[...], 0.0)
        y2_o[...] = jnp.dot(
            h1, w2_r[...], preferred_element_type=jnp.float32) * dis_r[...]

    return pl.pallas_call(
        body,
        grid=(np_pad // TB,),
        in_specs=[_row_spec(DEPTH), _row_spec(DEPTH), _row_spec(DEPTH),
                  _row_spec(1),
                  _full_spec((DEPTH, 16)), _full_spec((1, 16)),
                  _full_spec((16, DEPTH))],
        out_specs=_row_spec(DEPTH),
        out_shape=jax.ShapeDtypeStruct((np_pad, DEPTH), jnp.float32),
    )(a0, a1, y1, dis, W1p, b1, W2p)


def _tc_c(np_pad, g0, g1, y2, dis, b2):
    def body(g0_r, g1_r, y2_r, dis_r, b2_r, out_o):
        t = (g0_r[...] + g1_r[...] + y2_r[...]) * dis_r[...]
        out_o[...] = t[:, :2] + b2_r[...]

    return pl.pallas_call(
        body,
        grid=(np_pad // TB,),
        in_specs=[_row_spec(DEPTH), _row_spec(DEPTH), _row_spec(DEPTH),
                  _row_spec(1), _full_spec((1, 2))],
        out_specs=_row_spec(2),
        out_shape=jax.ShapeDtypeStruct((np_pad, 2), jnp.float32),
    )(g0, g1, y2, dis, b2)


def kernel(x, edge_index, W1, b1, W2, b2):
    n_nodes = x.shape[0]
    n_edges = edge_index.shape[1]
    np_pad, nblk = _geom(n_nodes, n_edges)
    e_pad = nblk * CHUNK

    fill = jnp.full((e_pad - n_edges,), n_nodes, dtype=jnp.int32)
    src2 = jnp.concatenate([edge_index[0], fill]).reshape(-1, LANES)
    dst2 = jnp.concatenate([edge_index[1], fill]).reshape(-1, LANES)
    xp = jnp.pad(x, ((0, np_pad - n_nodes), (0, DEPTH - x.shape[1])))
    ones_k = jnp.ones((LANES, DEPTH), jnp.float32)
    zeros_t = jnp.zeros((np_pad, DEPTH), jnp.float32)
    W1p = jnp.pad(W1, ((0, DEPTH - W1.shape[0]), (0, 0)))
    W2p = jnp.pad(W2, ((0, 0), (0, DEPTH - W2.shape[1])))

    deg = _sc_scatter_kernel(np_pad, nblk, False)(dst2, ones_k, zeros_t)
    dis, y1 = _tc_a(np_pad, deg[0][:, :1], deg[1][:, :1], xp)
    agg1 = _sc_scatter_kernel(np_pad, nblk, True)(src2, dst2, y1, zeros_t)
    y2 = _tc_b(np_pad, agg1[0], agg1[1], y1, dis,
               W1p, b1.reshape(1, 16), W2p)
    agg2 = _sc_scatter_kernel(np_pad, nblk, True)(src2, dst2, y2, zeros_t)
    out = _tc_c(np_pad, agg2[0], agg2[1], y2, dis, b2.reshape(1, 2))
    return out[:n_nodes]
